# baseline (speedup 1.0000x reference)
import functools

import jax
import jax.numpy as jnp
from jax import lax
from jax.experimental import pallas as pl
from jax.experimental.pallas import tpu as pltpu
from jax.experimental.pallas import tpu_sc as plsc

BATCH = 16
TOTAL_TOKENS = 32768
D_IN = 512
D_OUT = 64

NC = 2
NS = 16
NW = NC * NS
NSPLIT = 2
T_SPLIT = TOTAL_TOKENS // NSPLIT
ROWS_PER_W = T_SPLIT // NW

MM_BLK = 2048


def _tc_matmul(flat_ref, w_ref, y_ref):
    y_ref[...] = lax.dot_general(
        w_ref[...], flat_ref[...], (((1,), (1,)), ((), ())),
        preferred_element_type=jnp.float32,
    )


_sc_mesh = plsc.VectorSubcoreMesh(
    core_axis_name="c", subcore_axis_name="s", num_cores=NC, num_subcores=NS
)


def _make_sc_segment_sums(h):
    @functools.partial(
        pl.kernel,
        out_type=(
            jax.ShapeDtypeStruct((NW, BATCH, D_OUT * 16), jnp.float32),
            jax.ShapeDtypeStruct((NW, BATCH, 16), jnp.float32),
        ),
        mesh=_sc_mesh,
        scratch_types=[
            pltpu.VMEM((ROWS_PER_W,), jnp.int32),
            pltpu.VMEM((D_OUT, ROWS_PER_W), jnp.float32),
            pltpu.VMEM((BATCH, D_OUT * 16), jnp.float32),
            pltpu.VMEM((BATCH, 16), jnp.float32),
            pltpu.VMEM((D_OUT, 16), jnp.float32),
        ],
        name=f"sc_segment_sums_h{h}",
        compiler_params=pltpu.CompilerParams(needs_layout_passes=False),
    )
    def _sc_segment_sums(y_hbm, seg_hbm, out_sum, out_cnt,
                         idx_v, buf, acc, accc, run):
        cid = lax.axis_index("c")
        sid = lax.axis_index("s")
        wid = sid * NC + cid

        pltpu.sync_copy(
            seg_hbm.at[pl.ds(h * T_SPLIT + wid * ROWS_PER_W, ROWS_PER_W)],
            idx_v)
        pltpu.sync_copy(y_hbm.at[:, pl.ds(wid * ROWS_PER_W, ROWS_PER_W)], buf)

        zero = jnp.zeros((16,), jnp.float32)

        def _zero(i, _):
            for k in range(D_OUT):
                acc[i, pl.ds(k * 16, 16)] = zero
            accc[i, :] = zero
            return 0

        lax.fori_loop(0, BATCH, _zero, 0)
        for c in range(D_OUT):
            run[c, :] = zero

        def _grp(g, prev):
            lanes = lax.iota(jnp.int32, 16)
            ones = jnp.ones((16,), jnp.float32)
            r0 = g * 16
            ids16 = idx_v[pl.ds(r0, 16)]
            plsc.addupdate_scatter(accc, [ids16, lanes], ones)
            umin = jnp.min(ids16)
            umax = jnp.max(ids16)

            def _fast(prev):
                for c in range(D_OUT):
                    run[c, :] = run[c, :] + buf[c, pl.ds(r0, 16)]
                return prev

            def _slow(prev):
                pv = lanes * 0 + prev
                z16 = jnp.zeros((16,), jnp.float32)
                for c in range(D_OUT):
                    plsc.addupdate_scatter(acc, [pv, lanes + c * 16],
                                           run[c, :])
                    run[c, :] = z16
                    plsc.addupdate_scatter(acc, [ids16, lanes + c * 16],
                                           buf[c, pl.ds(r0, 16)])
                return umax

            return lax.cond((umin == prev) & (umax == prev),
                            _fast, _slow, prev)

        prev0 = jnp.min(idx_v[pl.ds(0, 16)])
        prev_last = lax.fori_loop(0, ROWS_PER_W // 16, _grp, prev0)

        lanes_f = lax.iota(jnp.int32, 16)
        pv_f = lanes_f * 0 + prev_last
        for c in range(D_OUT):
            plsc.addupdate_scatter(acc, [pv_f, lanes_f + c * 16], run[c, :])

        pltpu.sync_copy(acc, out_sum.at[wid])
        pltpu.sync_copy(accc, out_cnt.at[wid])

    return _sc_segment_sums


_sc_calls = [_make_sc_segment_sums(h) for h in range(NSPLIT)]


def _tc_finish(ps0_ref, ps1_ref, pc0_ref, pc1_ref, b_ref, o_ref):
    pre = (jnp.sum(ps0_ref[...], axis=0)
           + jnp.sum(ps1_ref[...], axis=0))
    fold = (lax.broadcasted_iota(jnp.int32, (D_OUT * 16, D_OUT), 0) // 16
            == lax.broadcasted_iota(jnp.int32, (D_OUT * 16, D_OUT), 1)
            ).astype(jnp.float32)
    sums = lax.dot_general(pre, fold, (((1,), (0,)), ((), ())),
                           preferred_element_type=jnp.float32)
    cnts = jnp.sum(pc0_ref[...], axis=0) + jnp.sum(pc1_ref[...], axis=0)
    cnt = jnp.sum(cnts, axis=1, keepdims=True)
    avg = sums / jnp.maximum(cnt, 1.0)
    o_ref[...] = avg + b_ref[...]


def _matmul_call(flat, W, h):
    nblk = T_SPLIT // MM_BLK
    return pl.pallas_call(
        _tc_matmul,
        grid=(nblk,),
        in_specs=[
            pl.BlockSpec((MM_BLK, D_IN), lambda i, h=h: (h * nblk + i, 0)),
            pl.BlockSpec((D_OUT, D_IN), lambda i: (0, 0)),
        ],
        out_specs=pl.BlockSpec((D_OUT, MM_BLK), lambda i: (0, i)),
        out_shape=jax.ShapeDtypeStruct((D_OUT, T_SPLIT), jnp.float32),
    )(flat, W)


def kernel(flat, segment_ids, W, b):
    seg = segment_ids.astype(jnp.int32)
    partials = []
    for h in range(NSPLIT):
        y_h = _matmul_call(flat, W, h)
        partials.append(_sc_calls[h](y_h, seg))
    (ps0, pc0), (ps1, pc1) = partials
    out = pl.pallas_call(
        _tc_finish,
        out_shape=jax.ShapeDtypeStruct((BATCH, D_OUT), jnp.float32),
    )(ps0, ps1, pc0, pc1, b.reshape(1, D_OUT))
    return out


# speedup vs baseline: 1.1230x; 1.1230x over previous
import functools

import jax
import jax.numpy as jnp
from jax import lax
from jax.experimental import pallas as pl
from jax.experimental.pallas import tpu as pltpu
from jax.experimental.pallas import tpu_sc as plsc

BATCH = 16
TOTAL_TOKENS = 32768
D_IN = 512
D_OUT = 64

NC = 2
NS = 16
NW = NC * NS
NSPLIT = 2
T_SPLIT = TOTAL_TOKENS // NSPLIT
ROWS_PER_W = T_SPLIT // NW

MM_BLK = 4096


def _tc_matmul(flat_ref, w_ref, y_ref):
    y_ref[...] = lax.dot_general(
        w_ref[...], flat_ref[...], (((1,), (1,)), ((), ())),
        preferred_element_type=jnp.float32,
    )


_sc_mesh = plsc.VectorSubcoreMesh(
    core_axis_name="c", subcore_axis_name="s", num_cores=NC, num_subcores=NS
)


def _make_sc_segment_sums(h):
    @functools.partial(
        pl.kernel,
        out_type=(
            jax.ShapeDtypeStruct((NW, BATCH, D_OUT * 16), jnp.float32),
            jax.ShapeDtypeStruct((NW, BATCH, 16), jnp.float32),
        ),
        mesh=_sc_mesh,
        scratch_types=[
            pltpu.VMEM((ROWS_PER_W,), jnp.int32),
            pltpu.VMEM((D_OUT, ROWS_PER_W), jnp.float32),
            pltpu.VMEM((BATCH, D_OUT * 16), jnp.float32),
            pltpu.VMEM((BATCH, 16), jnp.float32),
        ],
        name=f"sc_segment_sums_h{h}",
        compiler_params=pltpu.CompilerParams(needs_layout_passes=False),
    )
    def _sc_segment_sums(y_hbm, seg_hbm, out_sum, out_cnt,
                         idx_v, buf, acc, accc):
        cid = lax.axis_index("c")
        sid = lax.axis_index("s")
        wid = sid * NC + cid

        pltpu.sync_copy(
            seg_hbm.at[pl.ds(h * T_SPLIT + wid * ROWS_PER_W, ROWS_PER_W)],
            idx_v)
        pltpu.sync_copy(y_hbm.at[:, pl.ds(wid * ROWS_PER_W, ROWS_PER_W)], buf)

        zero = jnp.zeros((16,), jnp.float32)

        def _zero(i, _):
            for k in range(D_OUT):
                acc[i, pl.ds(k * 16, 16)] = zero
            accc[i, :] = zero
            return 0

        lax.fori_loop(0, BATCH, _zero, 0)

        def _grp(g, _):
            lanes = lax.iota(jnp.int32, 16)
            ones = jnp.ones((16,), jnp.float32)
            r0 = g * 16
            ids16 = idx_v[pl.ds(r0, 16)]
            plsc.addupdate_scatter(accc, [ids16, lanes], ones)
            for c in range(D_OUT):
                vals = buf[c, pl.ds(r0, 16)]
                plsc.addupdate_scatter(acc, [ids16, lanes + c * 16], vals)
            return 0

        lax.fori_loop(0, ROWS_PER_W // 16, _grp, 0)

        pltpu.sync_copy(acc, out_sum.at[wid])
        pltpu.sync_copy(accc, out_cnt.at[wid])

    return _sc_segment_sums


_sc_calls = [_make_sc_segment_sums(h) for h in range(NSPLIT)]


def _tc_finish(ps0_ref, ps1_ref, pc0_ref, pc1_ref, b_ref, o_ref):
    pre = (jnp.sum(ps0_ref[...], axis=0)
           + jnp.sum(ps1_ref[...], axis=0))
    fold = (lax.broadcasted_iota(jnp.int32, (D_OUT * 16, D_OUT), 0) // 16
            == lax.broadcasted_iota(jnp.int32, (D_OUT * 16, D_OUT), 1)
            ).astype(jnp.float32)
    sums = lax.dot_general(pre, fold, (((1,), (0,)), ((), ())),
                           preferred_element_type=jnp.float32)
    cnts = jnp.sum(pc0_ref[...], axis=0) + jnp.sum(pc1_ref[...], axis=0)
    cnt = jnp.sum(cnts, axis=1, keepdims=True)
    avg = sums / jnp.maximum(cnt, 1.0)
    o_ref[...] = avg + b_ref[...]


def _matmul_call(flat, W, h):
    nblk = T_SPLIT // MM_BLK
    return pl.pallas_call(
        _tc_matmul,
        grid=(nblk,),
        in_specs=[
            pl.BlockSpec((MM_BLK, D_IN), lambda i, h=h: (h * nblk + i, 0)),
            pl.BlockSpec((D_OUT, D_IN), lambda i: (0, 0)),
        ],
        out_specs=pl.BlockSpec((D_OUT, MM_BLK), lambda i: (0, i)),
        out_shape=jax.ShapeDtypeStruct((D_OUT, T_SPLIT), jnp.float32),
    )(flat, W)


def kernel(flat, segment_ids, W, b):
    seg = segment_ids.astype(jnp.int32)
    partials = []
    for h in range(NSPLIT):
        y_h = _matmul_call(flat, W, h)
        partials.append(_sc_calls[h](y_h, seg))
    (ps0, pc0), (ps1, pc1) = partials
    out = pl.pallas_call(
        _tc_finish,
        out_shape=jax.ShapeDtypeStruct((BATCH, D_OUT), jnp.float32),
    )(ps0, ps1, pc0, pc1, b.reshape(1, D_OUT))
    return out


# speedup vs baseline: 1.1438x; 1.0185x over previous
import functools

import jax
import jax.numpy as jnp
from jax import lax
from jax.experimental import pallas as pl
from jax.experimental.pallas import tpu as pltpu
from jax.experimental.pallas import tpu_sc as plsc

BATCH = 16
TOTAL_TOKENS = 32768
D_IN = 512
D_OUT = 64

NC = 2
NS = 16
NW = NC * NS
NSPLIT = 2
T_SPLIT = TOTAL_TOKENS // NSPLIT
ROWS_PER_W = T_SPLIT // NW

MM_BLK = 4096


def _tc_matmul(flat_ref, w_ref, y_ref):
    y_ref[...] = lax.dot_general(
        w_ref[...], flat_ref[...], (((1,), (1,)), ((), ())),
        preferred_element_type=jnp.float32,
    )


_sc_mesh = plsc.VectorSubcoreMesh(
    core_axis_name="c", subcore_axis_name="s", num_cores=NC, num_subcores=NS
)


def _make_sc_segment_sums(h):
    @functools.partial(
        pl.kernel,
        out_type=(
            jax.ShapeDtypeStruct((NW, BATCH, D_OUT * 16), jnp.float32),
            jax.ShapeDtypeStruct((NW, BATCH, 16), jnp.float32),
        ),
        mesh=_sc_mesh,
        scratch_types=[
            pltpu.VMEM((ROWS_PER_W,), jnp.int32),
            pltpu.VMEM((D_OUT, ROWS_PER_W), jnp.float32),
            pltpu.VMEM((BATCH, D_OUT * 8), jnp.float32),
            pltpu.VMEM((BATCH, D_OUT * 8), jnp.float32),
            pltpu.VMEM((BATCH, 16), jnp.float32),
        ],
        name=f"sc_segment_sums_h{h}",
        compiler_params=pltpu.CompilerParams(needs_layout_passes=False),
    )
    def _sc_segment_sums(y_hbm, seg_hbm, out_sum, out_cnt,
                         idx_v, buf, acc_a, acc_b, accc):
        cid = lax.axis_index("c")
        sid = lax.axis_index("s")
        wid = sid * NC + cid

        pltpu.sync_copy(
            seg_hbm.at[pl.ds(h * T_SPLIT + wid * ROWS_PER_W, ROWS_PER_W)],
            idx_v)
        pltpu.sync_copy(y_hbm.at[:, pl.ds(wid * ROWS_PER_W, ROWS_PER_W)], buf)

        zero = jnp.zeros((16,), jnp.float32)

        def _zero(i, _):
            for k in range(D_OUT // 2):
                acc_a[i, pl.ds(k * 16, 16)] = zero
                acc_b[i, pl.ds(k * 16, 16)] = zero
            accc[i, :] = zero
            return 0

        lax.fori_loop(0, BATCH, _zero, 0)

        def _grp(g, _):
            lanes = lax.iota(jnp.int32, 16)
            ones = jnp.ones((16,), jnp.float32)
            r0 = g * 16
            ids16 = idx_v[pl.ds(r0, 16)]
            plsc.addupdate_scatter(accc, [ids16, lanes], ones)
            for c in range(D_OUT):
                vals = buf[c, pl.ds(r0, 16)]
                tgt = acc_a if c % 2 == 0 else acc_b
                plsc.addupdate_scatter(tgt, [ids16, lanes + (c // 2) * 16],
                                       vals)
            return 0

        lax.fori_loop(0, ROWS_PER_W // 16, _grp, 0)

        pltpu.sync_copy(acc_a, out_sum.at[wid, :, pl.ds(0, D_OUT * 8)])
        pltpu.sync_copy(acc_b, out_sum.at[wid, :, pl.ds(D_OUT * 8, D_OUT * 8)])
        pltpu.sync_copy(accc, out_cnt.at[wid])

    return _sc_segment_sums


_sc_calls = [_make_sc_segment_sums(h) for h in range(NSPLIT)]


def _tc_finish(ps0_ref, ps1_ref, pc0_ref, pc1_ref, b_ref, o_ref):
    pre = (jnp.sum(ps0_ref[...], axis=0)
           + jnp.sum(ps1_ref[...], axis=0))
    kk = lax.broadcasted_iota(jnp.int32, (D_OUT * 16, D_OUT), 0)
    cc = lax.broadcasted_iota(jnp.int32, (D_OUT * 16, D_OUT), 1)
    half = D_OUT * 8
    fold = (((kk < half) & (cc == 2 * (kk // 16)))
            | ((kk >= half) & (cc == 2 * ((kk - half) // 16) + 1))
            ).astype(jnp.float32)
    sums = lax.dot_general(pre, fold, (((1,), (0,)), ((), ())),
                           preferred_element_type=jnp.float32)
    cnts = jnp.sum(pc0_ref[...], axis=0) + jnp.sum(pc1_ref[...], axis=0)
    cnt = jnp.sum(cnts, axis=1, keepdims=True)
    avg = sums / jnp.maximum(cnt, 1.0)
    o_ref[...] = avg + b_ref[...]


def _matmul_call(flat, W, h):
    nblk = T_SPLIT // MM_BLK
    return pl.pallas_call(
        _tc_matmul,
        grid=(nblk,),
        in_specs=[
            pl.BlockSpec((MM_BLK, D_IN), lambda i, h=h: (h * nblk + i, 0)),
            pl.BlockSpec((D_OUT, D_IN), lambda i: (0, 0)),
        ],
        out_specs=pl.BlockSpec((D_OUT, MM_BLK), lambda i: (0, i)),
        out_shape=jax.ShapeDtypeStruct((D_OUT, T_SPLIT), jnp.float32),
    )(flat, W)


def kernel(flat, segment_ids, W, b):
    seg = segment_ids.astype(jnp.int32)
    partials = []
    for h in range(NSPLIT):
        y_h = _matmul_call(flat, W, h)
        partials.append(_sc_calls[h](y_h, seg))
    (ps0, pc0), (ps1, pc1) = partials
    out = pl.pallas_call(
        _tc_finish,
        out_shape=jax.ShapeDtypeStruct((BATCH, D_OUT), jnp.float32),
    )(ps0, ps1, pc0, pc1, b.reshape(1, D_OUT))
    return out
